# baseline (speedup 1.0000x reference)
import jax
import jax.numpy as jnp
from jax import lax
from jax.experimental import pallas as pl
from jax.experimental.pallas import tpu as pltpu
from jax.experimental.pallas import tpu_sc as plsc

B, S, D = 4, 8192, 768
NC, NS = 2, 16
NW = NC * NS
LANES = 16
ROWS_PER_W = S // NW
R = 32
CHUNK = R * D
N_CHUNKS = ROWS_PER_W // R


def _sc_body(x_hbm, pe_hbm, out_hbm, pe_v, x_v):
    wid = lax.axis_index("s") * NC + lax.axis_index("c")
    base = wid * (ROWS_PER_W * D)
    for ci in range(N_CHUNKS):
        pe_off = base + ci * CHUNK
        pltpu.sync_copy(pe_hbm.at[pl.ds(pe_off, CHUNK)], pe_v)
        for b in range(B):
            x_off = b * (S * D) + pe_off
            pltpu.sync_copy(x_hbm.at[pl.ds(x_off, CHUNK)], x_v)

            @plsc.parallel_loop(0, CHUNK, LANES, unroll=8)
            def _(i):
                x_v[pl.ds(i, LANES)] += pe_v[pl.ds(i, LANES)]

            pltpu.sync_copy(x_v, out_hbm.at[pl.ds(x_off, CHUNK)])


def kernel(x, pe_weight):
    mesh = plsc.VectorSubcoreMesh(core_axis_name="c", subcore_axis_name="s")
    f = pl.kernel(
        _sc_body,
        out_type=jax.ShapeDtypeStruct((B * S * D,), jnp.float32),
        mesh=mesh,
        scratch_types=[
            pltpu.VMEM((CHUNK,), jnp.float32),
            pltpu.VMEM((CHUNK,), jnp.float32),
        ],
    )
    out = f(x.reshape(-1), pe_weight.reshape(-1))
    return out.reshape(B, S, D)


# speedup vs baseline: 1.2307x; 1.2307x over previous
import jax
import jax.numpy as jnp
from jax import lax
from jax.experimental import pallas as pl
from jax.experimental.pallas import tpu as pltpu
from jax.experimental.pallas import tpu_sc as plsc

B, S, D = 4, 8192, 768
NC, NS = 2, 16
NW = NC * NS
LANES = 16
ROWS_PER_W = S // NW
R = 32
CHUNK = R * D
N_CHUNKS = ROWS_PER_W // R
NBUF = 3
PF = 2
T = N_CHUNKS * B


def _sc_body(x_hbm, pe_hbm, out_hbm,
             x_v0, x_v1, x_v2, pe_v0, pe_v1,
             si0, si1, si2, so0, so1, so2, sp0, sp1):
    x_v = [x_v0, x_v1, x_v2]
    pe_v = [pe_v0, pe_v1]
    sem_in = [si0, si1, si2]
    sem_out = [so0, so1, so2]
    sem_pe = [sp0, sp1]
    wid = lax.axis_index("s") * NC + lax.axis_index("c")
    base = wid * (ROWS_PER_W * D)

    def x_off(t):
        ci, b = divmod(t, B)
        return b * (S * D) + base + ci * CHUNK

    in_h, out_h, pe_h = {}, {}, {}

    def issue_in(t):
        in_h[t] = pltpu.async_copy(
            x_hbm.at[pl.ds(x_off(t), CHUNK)], x_v[t % NBUF], sem_in[t % NBUF])

    def issue_pe(ci):
        pe_h[ci] = pltpu.async_copy(
            pe_hbm.at[pl.ds(base + ci * CHUNK, CHUNK)], pe_v[ci % 2],
            sem_pe[ci % 2])

    issue_pe(0)
    if N_CHUNKS > 1:
        issue_pe(1)
    for t in range(PF):
        issue_in(t)

    for t in range(T):
        ci, b = divmod(t, B)
        slot = t % NBUF
        if b == 0:
            pe_h[ci].wait()
        in_h[t].wait()

        xb, pb = x_v[slot], pe_v[ci % 2]

        @plsc.parallel_loop(0, CHUNK, LANES, unroll=8)
        def _(i):
            xb[pl.ds(i, LANES)] += pb[pl.ds(i, LANES)]

        out_h[t] = pltpu.async_copy(
            xb, out_hbm.at[pl.ds(x_off(t), CHUNK)], sem_out[slot])

        if b == B - 1 and ci + 2 < N_CHUNKS:
            issue_pe(ci + 2)
        nt = t + PF
        if nt < T:
            if nt >= NBUF:
                out_h[nt - NBUF].wait()
            issue_in(nt)

    for t in range(T - NBUF, T):
        out_h[t].wait()


def kernel(x, pe_weight):
    mesh = plsc.VectorSubcoreMesh(core_axis_name="c", subcore_axis_name="s")
    f = pl.kernel(
        _sc_body,
        out_type=jax.ShapeDtypeStruct((B * S * D,), jnp.float32),
        mesh=mesh,
        scratch_types=(
            [pltpu.VMEM((CHUNK,), jnp.float32)] * (NBUF + 2)
            + [pltpu.SemaphoreType.DMA] * (2 * NBUF + 2)
        ),
    )
    out = f(x.reshape(-1), pe_weight.reshape(-1))
    return out.reshape(B, S, D)


# speedup vs baseline: 3.6975x; 3.0044x over previous
import jax
import jax.numpy as jnp
from jax import lax
from jax.experimental import pallas as pl
from jax.experimental.pallas import tpu as pltpu
from jax.experimental.pallas import tpu_sc as plsc

B, S, D = 4, 8192, 768
NC, NS = 2, 16
NW = NC * NS
LANES = 16
ROWS_PER_W = S // NW
R = 32
N_CHUNKS = ROWS_PER_W // R
NBUF = 3
PF = 2
T = N_CHUNKS * B


def _sc_body(x_hbm, pe_hbm, out_hbm,
             x_v0, x_v1, x_v2, pe_v0, pe_v1,
             si0, si1, si2, so0, so1, so2, sp0, sp1):
    x_v = [x_v0, x_v1, x_v2]
    pe_v = [pe_v0, pe_v1]
    sem_in = [si0, si1, si2]
    sem_out = [so0, so1, so2]
    sem_pe = [sp0, sp1]
    wid = lax.axis_index("s") * NC + lax.axis_index("c")
    base = wid * ROWS_PER_W

    in_h, out_h, pe_h = {}, {}, {}

    def issue_in(t):
        ci, b = divmod(t, B)
        in_h[t] = pltpu.async_copy(
            x_hbm.at[b, pl.ds(base + ci * R, R)], x_v[t % NBUF],
            sem_in[t % NBUF])

    def issue_pe(ci):
        pe_h[ci] = pltpu.async_copy(
            pe_hbm.at[pl.ds(base + ci * R, R)], pe_v[ci % 2], sem_pe[ci % 2])

    issue_pe(0)
    if N_CHUNKS > 1:
        issue_pe(1)
    for t in range(PF):
        issue_in(t)

    for t in range(T):
        ci, b = divmod(t, B)
        slot = t % NBUF
        if b == 0:
            pe_h[ci].wait()
        in_h[t].wait()

        xb, pb = x_v[slot], pe_v[ci % 2]

        @plsc.parallel_loop(0, R, 1)
        def _(r):
            @plsc.parallel_loop(0, D, LANES, unroll=8)
            def _(c):
                xb[r, pl.ds(c, LANES)] += pb[r, pl.ds(c, LANES)]

        out_h[t] = pltpu.async_copy(
            xb, out_hbm.at[b, pl.ds(base + ci * R, R)], sem_out[slot])

        if b == B - 1 and ci + 2 < N_CHUNKS:
            issue_pe(ci + 2)
        nt = t + PF
        if nt < T:
            if nt >= NBUF:
                out_h[nt - NBUF].wait()
            issue_in(nt)

    for t in range(T - NBUF, T):
        out_h[t].wait()


def kernel(x, pe_weight):
    mesh = plsc.VectorSubcoreMesh(core_axis_name="c", subcore_axis_name="s")
    f = pl.kernel(
        _sc_body,
        out_type=jax.ShapeDtypeStruct((B, S, D), jnp.float32),
        mesh=mesh,
        compiler_params=pltpu.CompilerParams(use_tc_tiling_on_sc=True),
        scratch_types=(
            [pltpu.VMEM((R, D), jnp.float32)] * (NBUF + 2)
            + [pltpu.SemaphoreType.DMA] * (2 * NBUF + 2)
        ),
    )
    return f(x, pe_weight)
